# baseline (speedup 1.0000x reference)
import functools

import jax
import jax.numpy as jnp
from jax import lax
from jax.experimental import pallas as pl
from jax.experimental.pallas import tpu as pltpu
from jax.experimental.pallas import tpu_sc as plsc

GRID_H = 1024
GRID_W = 1024
BATCH = 16384
LANES = 16

_info = plsc.get_sparse_core_info()
_NC = _info.num_cores
_NS = _info.num_subcores
_NW = _NC * _NS
_PTS = BATCH // _NW
_NVREG = _PTS // LANES
_NBLK = 8
_BV = _NVREG // _NBLK
_BIDX = 4 * _BV * LANES


def _sc_body(xy_hbm, grid_hbm, s_hbm, o_hbm, out_hbm,
             xyv, wxv, wyv, outv, sov, iall, gall,
             sem_in, sem_a, sem_b):
    wid = lax.axis_index("s") * _NC + lax.axis_index("c")
    base = wid * _PTS
    cin0 = pltpu.async_copy(xy_hbm.at[:, pl.ds(base, _PTS)], xyv, sem_in)
    cin1 = pltpu.async_copy(s_hbm, sov.at[pl.ds(0, 1)], sem_in)
    cin2 = pltpu.async_copy(o_hbm, sov.at[pl.ds(8, 1)], sem_in)
    cin0.wait()

    def compute(g, carry):
        i = pl.multiple_of(g * LANES, LANES)
        xx = xyv[0, pl.ds(i, LANES)]
        yy = xyv[1, pl.ds(i, LANES)]
        xf = jnp.minimum(jnp.maximum(xx, 0.0), 1.0) * float(GRID_W - 1)
        yf = jnp.minimum(jnp.maximum(yy, 0.0), 1.0) * float(GRID_H - 1)
        x0 = xf.astype(jnp.int32)
        y0 = yf.astype(jnp.int32)
        wxv[pl.ds(i, LANES)] = xf - x0.astype(jnp.float32)
        wyv[pl.ds(i, LANES)] = yf - y0.astype(jnp.float32)
        dx = jnp.minimum(x0 + 1, GRID_W - 1) - x0
        r0 = y0 * GRID_W + x0
        r1 = jnp.minimum(y0 + 1, GRID_H - 1) * GRID_W + x0
        q = 4 * i
        iall[pl.ds(q, LANES)] = r0
        iall[pl.ds(q + LANES, LANES)] = r0 + dx
        iall[pl.ds(q + 2 * LANES, LANES)] = r1
        iall[pl.ds(q + 3 * LANES, LANES)] = r1 + dx
        return carry

    def make_blend(so, oo):
        def blend(g, carry):
            i = pl.multiple_of(g * LANES, LANES)
            q = 4 * i
            a00 = gall[pl.ds(q, LANES)]
            a01 = gall[pl.ds(q + LANES, LANES)]
            a10 = gall[pl.ds(q + 2 * LANES, LANES)]
            a11 = gall[pl.ds(q + 3 * LANES, LANES)]
            wx = wxv[pl.ds(i, LANES)]
            wy = wyv[pl.ds(i, LANES)]
            top = a00 + wx * (a01 - a00)
            bot = a10 + wx * (a11 - a10)
            val = top + wy * (bot - top)
            outv[pl.ds(i, LANES)] = val * so + oo
            return carry

        return blend

    sems = [sem_a, sem_b]
    copies = []
    for b in range(_NBLK):
        lax.fori_loop(b * _BV, (b + 1) * _BV, compute, 0, unroll=4)
        copies.append(pltpu.async_copy(
            grid_hbm.at[iall.at[pl.ds(b * _BIDX, _BIDX)]],
            gall.at[pl.ds(b * _BIDX, _BIDX)], sems[b % 2]))

    cin1.wait()
    cin2.wait()
    sovec = sov[...]
    so = jnp.broadcast_to(sovec[0], (LANES,))
    oo = jnp.broadcast_to(sovec[8], (LANES,))
    blend = make_blend(so, oo)
    for b in range(_NBLK):
        copies[b].wait()
        lax.fori_loop(b * _BV, (b + 1) * _BV, blend, 0, unroll=4)
        if b == _NBLK // 2 - 1:
            co_a = pltpu.async_copy(outv.at[pl.ds(0, _PTS // 2)],
                                    out_hbm.at[pl.ds(base, _PTS // 2)], sem_in)
    co_a.wait()
    pltpu.sync_copy(outv.at[pl.ds(_PTS // 2, _PTS // 2)],
                    out_hbm.at[pl.ds(base + _PTS // 2, _PTS // 2)])


_bilinear_sc = functools.partial(
    pl.kernel,
    out_type=jax.ShapeDtypeStruct((BATCH,), jnp.float32),
    mesh=plsc.VectorSubcoreMesh(core_axis_name="c", subcore_axis_name="s"),
    scratch_types=[
        pltpu.VMEM((2, _PTS), jnp.float32),
        pltpu.VMEM((_PTS,), jnp.float32),
        pltpu.VMEM((_PTS,), jnp.float32),
        pltpu.VMEM((_PTS,), jnp.float32),
        pltpu.VMEM((LANES,), jnp.float32),
        pltpu.VMEM((4 * _PTS,), jnp.int32),
        pltpu.VMEM((4 * _PTS,), jnp.float32),
        pltpu.SemaphoreType.DMA,
        pltpu.SemaphoreType.DMA,
        pltpu.SemaphoreType.DMA,
    ],
)(_sc_body)


def kernel(xy, grid, scale, offset):
    xyT = xy.T
    gflat = grid.reshape(-1)
    s1 = jnp.asarray(scale, jnp.float32).reshape(1)
    o1 = jnp.asarray(offset, jnp.float32).reshape(1)
    return _bilinear_sc(xyT, gflat, s1, o1)
